# Initial kernel scaffold; baseline (speedup 1.0000x reference)
import jax
import jax.numpy as jnp
from jax.experimental import pallas as pl


def kernel(llr_demapper):
    raise NotImplementedError("write your pallas kernel here")


# speedup vs baseline: 8.2087x; 8.2087x over previous
import functools

import numpy as np
import jax
import jax.numpy as jnp
from jax import lax
from jax.experimental import pallas as pl
from jax.experimental.pallas import tpu as pltpu
from jax.experimental.pallas import tpu_sc as plsc

_N = 6144
_NC = 3072
_DC = 6
_NE = 18432
_NB_ITER = 5
_CLIP = 1e-7
_B = 64
_NW = 32
_VPW = _N // _NW
_CH = 96
_NCH = _VPW // _CH


def _build_sigg() -> np.ndarray:
    n, dv, dc = _N, 3, _DC
    rows_per_block = n // dc
    rng = np.random.RandomState(0)
    base = np.zeros((rows_per_block, n), dtype=np.int8)
    for i in range(rows_per_block):
        base[i, i * dc:(i + 1) * dc] = 1
    blocks = [base]
    for _ in range(dv - 1):
        perm = rng.permutation(n)
        blocks.append(base[:, perm])
    H = np.concatenate(blocks, axis=0)
    edges = np.stack(np.where(H == 1), axis=1)
    vn = edges[:, 1].astype(np.int64)
    sigg = np.zeros((dv, n), dtype=np.int32)
    for k in range(dv):
        pi = vn[n * k:n * (k + 1)]
        sig = np.argsort(pi)
        e = n * k + sig
        sigg[k] = ((e % dc) * _NC + e // dc).astype(np.int32)
    return sigg


_SIGG = _build_sigg()


_sc_mesh = plsc.VectorSubcoreMesh(core_axis_name="c", subcore_axis_name="s")


@functools.partial(
    pl.kernel,
    out_type=(
        jax.ShapeDtypeStruct((_NE, 128), jnp.float32),
        jax.ShapeDtypeStruct((_N, 128), jnp.float32),
    ),
    mesh=_sc_mesh,
    scratch_types=[
        pltpu.VMEM((3 * _NCH, _CH), jnp.int32),
        pltpu.VMEM((_VPW, 128), jnp.float32),
    ],
)
def _sc_vn(l_hbm, cv_hbm, sigg_hbm, tt_hbm, tlin_hbm, idx_v, tbuf):
    wid = lax.axis_index("s") * 2 + lax.axis_index("c")
    base = wid * _VPW
    for k in range(3):
        for j in range(_NCH):
            pltpu.sync_copy(sigg_hbm.at[pl.ds(k * _N + base + j * _CH, _CH)],
                            idx_v.at[_NCH * k + j])
    pltpu.sync_copy(l_hbm.at[pl.ds(base, _VPW)], tbuf)
    for k in range(3):
        for j in range(_NCH):
            pltpu.sync_copy(cv_hbm.at[idx_v.at[_NCH * k + j]],
                            tbuf.at[pl.ds(j * _CH, _CH)], add=True)
    pltpu.sync_copy(tbuf, tlin_hbm.at[pl.ds(base, _VPW)])
    for k in range(3):
        for j in range(_NCH):
            pltpu.sync_copy(tbuf.at[pl.ds(j * _CH, _CH)],
                            tt_hbm.at[idx_v.at[_NCH * k + j]])


def _tc_cn_body(tt_ref, cv_ref, out_ref):
    t = [jnp.tanh(0.5 * (tt_ref[i] - cv_ref[i])) for i in range(_DC)]
    pre = [None] * _DC
    suf = [None] * _DC
    acc = None
    for i in range(_DC):
        pre[i] = acc
        acc = t[i] if acc is None else acc * t[i]
    acc = None
    for i in range(_DC - 1, -1, -1):
        suf[i] = acc
        acc = t[i] if acc is None else acc * t[i]
    for i in range(_DC):
        if pre[i] is None:
            loo = suf[i]
        elif suf[i] is None:
            loo = pre[i]
        else:
            loo = pre[i] * suf[i]
        y = loo - jnp.sign(loo) * _CLIP
        a = jnp.abs(y)
        out_ref[i] = jnp.sign(y) * jnp.log1p(2.0 * a / (1.0 - a))


_tc_cn = pl.pallas_call(
    _tc_cn_body,
    out_shape=jax.ShapeDtypeStruct((_DC, _NC, 128), jnp.float32),
)


def kernel(llr_demapper):
    L = (-llr_demapper).T.astype(jnp.float32)
    L = jnp.pad(L, ((0, 0), (0, 128 - _B)))
    sigg = jnp.asarray(_SIGG.reshape(-1))
    cv = jnp.zeros((_NE, 128), jnp.float32)
    for _ in range(_NB_ITER + 1):
        TT, _ = _sc_vn(L, cv, sigg)
        cv = _tc_cn(TT.reshape(_DC, _NC, 128),
                    cv.reshape(_DC, _NC, 128)).reshape(_NE, 128)
    _, T = _sc_vn(L, cv, sigg)
    return -(T[:, :_B].T)
